# baseline (speedup 1.0000x reference)
import jax
import jax.numpy as jnp
from jax.experimental import pallas as pl
from jax.experimental.pallas import tpu as pltpu

_MAX_BIN = 16
_SUB = 8
_RB = _SUB * 128
_UNROLL = 8
_GROUPS = 4


def _dp_kernel(x_ref, o_ref):
    t_total = x_ref.shape[1]
    zeros = jnp.zeros((_SUB, 128), jnp.float32)
    ones = jnp.ones((_SUB, 128), jnp.float32)
    init = (ones,) + (zeros,) * _MAX_BIN

    def body(i, dp):
        ps = x_ref[0, pl.ds(i * _UNROLL, _UNROLL)]
        for j in range(_UNROLL):
            p = ps[j]
            q = 1.0 - p
            new = [dp[0] * q]
            for k in range(1, _MAX_BIN + 1):
                new.append(dp[k] * q + dp[k - 1] * p)
            new[_MAX_BIN] = new[_MAX_BIN] + dp[_MAX_BIN]
            dp = tuple(new)
        return dp

    dp = jax.lax.fori_loop(0, t_total // _UNROLL, body, init)
    for k in range(_MAX_BIN + 1):
        o_ref[0, k] = dp[k]


def kernel(slice_probs) -> jnp.ndarray:
    B, T = slice_probs.shape
    nb = B // _RB
    groups = min(_GROUPS, nb)
    ng = nb // groups
    x5 = slice_probs.reshape(groups, ng, _SUB, 128, T)
    outs = []
    for g in range(groups):
        xt = jnp.transpose(x5[g], (0, 3, 1, 2))
        outs.append(pl.pallas_call(
            _dp_kernel,
            grid=(ng,),
            in_specs=[pl.BlockSpec((1, T, _SUB, 128), lambda i: (i, 0, 0, 0))],
            out_specs=pl.BlockSpec((1, _MAX_BIN + 1, _SUB, 128),
                                   lambda i: (i, 0, 0, 0)),
            out_shape=jax.ShapeDtypeStruct((ng, _MAX_BIN + 1, _SUB, 128),
                                           jnp.float32),
            compiler_params=pltpu.CompilerParams(
                dimension_semantics=("parallel",),
                vmem_limit_bytes=56 * 1024 * 1024,
            ),
            name="soft_count_dp",
        )(xt))
    out = jnp.stack(outs)
    return out.transpose(0, 1, 3, 4, 2).reshape(B, _MAX_BIN + 1)


# speedup vs baseline: 1.5599x; 1.5599x over previous
import jax
import jax.numpy as jnp
from jax.experimental import pallas as pl
from jax.experimental.pallas import tpu as pltpu

_MAX_BIN = 16
_RB = 1024
_TC = 256
_UNROLL = 8


def _dp_kernel(x_ref, o_ref, y_ref):
    t_total = x_ref.shape[3]
    tc = min(_TC, t_total)
    n_chunks = t_total // tc
    unroll = min(_UNROLL, tc)
    zeros = jnp.zeros((8, 128), jnp.float32)
    ones = jnp.ones((8, 128), jnp.float32)
    dp = (ones,) + (zeros,) * _MAX_BIN

    sub = jax.lax.broadcasted_iota(jnp.int32, (8, 128), 0)

    for c in range(n_chunks):
        par = c % 2
        zs = [jnp.transpose(x_ref[0, s, :, pl.ds(c * tc, tc)])
              for s in range(8)]
        for g in range(tc // 8):
            a = [zs[s][8 * g:8 * g + 8] for s in range(8)]
            for k in (1, 2, 4):
                b = list(a)
                for s in range(8):
                    shift = k if (s & k) == 0 else 8 - k
                    rolled = pltpu.roll(a[s ^ k], shift, axis=0)
                    b[s] = jnp.where(((sub ^ s) & k) == 0, a[s], rolled)
                a = b
            for tp in range(8):
                y_ref[par, 8 * g + tp] = a[tp]

        def body(i, dp):
            ps = y_ref[par, pl.ds(i * unroll, unroll)]
            for j in range(unroll):
                p = ps[j]
                q = 1.0 - p
                new = [dp[0] * q]
                for k in range(1, _MAX_BIN + 1):
                    new.append(dp[k] * q + dp[k - 1] * p)
                new[_MAX_BIN] = new[_MAX_BIN] + dp[_MAX_BIN]
                dp = tuple(new)
            return dp

        dp = jax.lax.fori_loop(0, tc // unroll, body, dp)

    for k in range(_MAX_BIN + 1):
        o_ref[0, k] = dp[k]


def kernel(slice_probs) -> jnp.ndarray:
    B, T = slice_probs.shape
    nb = B // _RB
    x4 = slice_probs.reshape(nb, 8, 128, T)
    out = pl.pallas_call(
        _dp_kernel,
        grid=(nb,),
        in_specs=[pl.BlockSpec((1, 8, 128, T), lambda i: (i, 0, 0, 0))],
        out_specs=pl.BlockSpec((1, _MAX_BIN + 1, 8, 128), lambda i: (i, 0, 0, 0)),
        out_shape=jax.ShapeDtypeStruct((nb, _MAX_BIN + 1, 8, 128), jnp.float32),
        scratch_shapes=[pltpu.VMEM((2, min(_TC, T), 8, 128), jnp.float32)],
        compiler_params=pltpu.CompilerParams(
            dimension_semantics=("parallel",),
            vmem_limit_bytes=56 * 1024 * 1024,
        ),
        name="soft_count_dp",
    )(x4)
    return out.transpose(0, 2, 3, 1).reshape(B, _MAX_BIN + 1)
